# baseline (speedup 1.0000x reference)
import functools

import jax
import jax.numpy as jnp
import numpy as np
from jax.experimental import pallas as pl

S = 2048
H = 12
D = 64
RADIUS_SQ = 6.25
BQ = 256
NEG = -1e30
SCALE = float(1.0 / np.sqrt(D))


def _attn_kernel(q_ref, k_ref, v_ref, qspan_ref, kspan_ref, caus_ref,
                 qc_ref, kc_ref, o_ref):
    i = pl.program_id(1)
    q = q_ref[0]
    k = k_ref[0]
    v = v_ref[0]

    s = jax.lax.dot_general(q, k, (((1,), (1,)), ((), ())),
                            preferred_element_type=jnp.float32) * SCALE

    qspan = qspan_ref[...]
    kspan = kspan_ref[...]
    caus = caus_ref[...]
    qx = qc_ref[:, 0:1]
    qy = qc_ref[:, 1:2]
    kx = kc_ref[0:1, :]
    ky = kc_ref[1:2, :]

    qidx = i * BQ + jax.lax.broadcasted_iota(jnp.int32, (BQ, 1), 0)
    kidx = jax.lax.broadcasted_iota(jnp.int32, (1, S), 1)

    dist = (qx - kx) ** 2 + (qy - ky) ** 2
    time_ok = (caus == 0) | (qidx >= kidx)
    mask = (kspan < qspan) | ((kspan == qspan) & time_ok & (dist < RADIUS_SQ))

    s = jnp.where(mask, s, NEG)
    m = jnp.max(s, axis=1, keepdims=True)
    p = jnp.exp(s - m)
    l = jnp.sum(p, axis=1, keepdims=True)
    o = jax.lax.dot_general(p, v, (((1,), (0,)), ((), ())),
                            preferred_element_type=jnp.float32)
    o_ref[0] = o / l


@functools.partial(jax.jit, static_argnames=())
def kernel(q, k, v, coords, span_ids, is_causal):
    q3 = q[0]
    k3 = k[0]
    v3 = v[0]
    span_col = span_ids.reshape(S, 1)
    span_row = span_ids.reshape(1, S)
    caus_col = is_causal.astype(jnp.int32).reshape(S, 1)
    coords_t = coords.T

    grid = (H, S // BQ)
    out = pl.pallas_call(
        _attn_kernel,
        grid=grid,
        in_specs=[
            pl.BlockSpec((1, BQ, D), lambda h, i: (h, i, 0)),
            pl.BlockSpec((1, S, D), lambda h, i: (h, 0, 0)),
            pl.BlockSpec((1, S, D), lambda h, i: (h, 0, 0)),
            pl.BlockSpec((BQ, 1), lambda h, i: (i, 0)),
            pl.BlockSpec((1, S), lambda h, i: (0, 0)),
            pl.BlockSpec((BQ, 1), lambda h, i: (i, 0)),
            pl.BlockSpec((BQ, 2), lambda h, i: (i, 0)),
            pl.BlockSpec((2, S), lambda h, i: (0, 0)),
        ],
        out_specs=pl.BlockSpec((1, BQ, D), lambda h, i: (h, i, 0)),
        out_shape=jax.ShapeDtypeStruct((H, S, D), jnp.float32),
    )(q3, k3, v3, span_col, span_row, caus_col, coords, coords_t)
    return out[None]
